# baseline (speedup 1.0000x reference)
import functools

import jax
import jax.numpy as jnp
from jax import lax
from jax.experimental import pallas as pl
from jax.experimental.pallas import tpu as pltpu
from jax.experimental.pallas import tpu_sc as plsc

B = 4096
L = 50
D = 64
NW = 32
BPW = B // NW
HALF_V = 50176


def _pool_kernel(ids_hbm, w_hbm, out_hbm, ids_tv, acc, acc_t, sem):
    wid = lax.axis_index("s") * 2 + lax.axis_index("c")
    pltpu.sync_copy(ids_hbm.at[:, pl.ds(wid * BPW, BPW)], ids_tv)

    zero = jnp.zeros((16,), jnp.float32)

    def zero_body(b, _):
        for d in range(4):
            acc[b, pl.ds(d * 16, 16)] = zero
        return 0

    lax.fori_loop(0, BPW, zero_body, 0)

    def fire_body(l, _):
        for g in range(BPW // 16):
            v = ids_tv[l, pl.ds(g * 16, 16)]
            v2 = jnp.where(v >= HALF_V, v * 2 - (2 * HALF_V - 1), v * 2)
            ids_tv[l, pl.ds(g * 16, 16)] = v2
        pltpu.async_copy(w_hbm.at[ids_tv.at[l]], acc, sem, add=True)
        return 0

    lax.fori_loop(0, L, fire_body, 0)

    invs = []
    for g in range(BPW // 16):
        def cnt_body(l, cnt):
            v = ids_tv[l, pl.ds(g * 16, 16)]
            return cnt + jnp.where(v != 0, 1.0, 0.0).astype(jnp.float32)

        cnt = lax.fori_loop(0, L, cnt_body, jnp.zeros((16,), jnp.float32))
        invs.append(1.0 / (cnt + 1e-8))

    def drain_body(l, _):
        pltpu.make_async_copy(w_hbm.at[ids_tv.at[0]], acc, sem).wait()
        return 0

    lax.fori_loop(0, L, drain_body, 0)

    lane = lax.iota(jnp.int32, 16)
    for d in range(D):
        col = jnp.full((16,), d, jnp.int32)
        for h in range(BPW // 16):
            v = plsc.load_gather(acc, [lane + h * 16, col])
            acc_t[d, pl.ds(h * 16, 16)] = v * invs[h]
    pltpu.sync_copy(acc_t, out_hbm.at[:, pl.ds(wid * BPW, BPW)])


V = 100000
_TC = 7168
_TCG = HALF_V // _TC


def _tc_transpose_body(xa_ref, xb_ref, y_ref):
    y_ref[:, 0:D] = xa_ref[...].T
    y_ref[:, D : 2 * D] = xb_ref[...].T


def _tc_transpose(wt):
    return pl.pallas_call(
        _tc_transpose_body,
        grid=(_TCG,),
        in_specs=[
            pl.BlockSpec((D, _TC), lambda j: (0, j)),
            pl.BlockSpec((D, _TC), lambda j: (0, j + _TCG)),
        ],
        out_specs=pl.BlockSpec((_TC, 2 * D), lambda j: (j, 0)),
        out_shape=jax.ShapeDtypeStruct((HALF_V, 2 * D), jnp.float32),
    )(wt, wt)


def kernel(ids, weight):
    ids_t = ids.astype(jnp.int32).T
    weight = _tc_transpose(weight.astype(jnp.float32).T).reshape(2 * HALF_V, D)
    mesh = plsc.VectorSubcoreMesh(core_axis_name="c", subcore_axis_name="s")
    run = functools.partial(
        pl.kernel,
        mesh=mesh,
        compiler_params=pltpu.CompilerParams(
            needs_layout_passes=False, use_tc_tiling_on_sc=False
        ),
        out_type=jax.ShapeDtypeStruct((D, B), jnp.float32),
        scratch_types=[
            pltpu.VMEM((L, BPW), jnp.int32),
            pltpu.VMEM((BPW, D), jnp.float32),
            pltpu.VMEM((D, BPW), jnp.float32),
            pltpu.SemaphoreType.DMA,
        ],
    )(_pool_kernel)
    return run(ids_t, weight).T


# speedup vs baseline: 1.1042x; 1.1042x over previous
import functools

import jax
import jax.numpy as jnp
from jax import lax
from jax.experimental import pallas as pl
from jax.experimental.pallas import tpu as pltpu
from jax.experimental.pallas import tpu_sc as plsc

B = 4096
L = 50
D = 64
NW = 32
BPW = B // NW
HALF_V = 50176


def _pool_kernel(ids_hbm, w_hbm, out_hbm, ids_tv, acc, inv_v, sem):
    wid = lax.axis_index("s") * 2 + lax.axis_index("c")
    pltpu.sync_copy(ids_hbm.at[:, pl.ds(wid * BPW, BPW)], ids_tv)

    zero = jnp.zeros((16,), jnp.float32)

    def zero_body(b, _):
        for d in range(4):
            acc[b, pl.ds(d * 16, 16)] = zero
        return 0

    lax.fori_loop(0, BPW, zero_body, 0)

    def fire_body(l, _):
        for g in range(BPW // 16):
            v = ids_tv[l, pl.ds(g * 16, 16)]
            v2 = jnp.where(v >= HALF_V, v * 2 - (2 * HALF_V - 1), v * 2)
            ids_tv[l, pl.ds(g * 16, 16)] = v2
        pltpu.async_copy(w_hbm.at[ids_tv.at[l]], acc, sem, add=True)
        return 0

    lax.fori_loop(0, L, fire_body, 0)

    for g in range(BPW // 16):
        def cnt_body(l, cnt):
            v = ids_tv[l, pl.ds(g * 16, 16)]
            return cnt + jnp.where(v != 0, 1.0, 0.0).astype(jnp.float32)

        cnt = lax.fori_loop(0, L, cnt_body, jnp.zeros((16,), jnp.float32))
        inv_v[pl.ds(g * 16, 16)] = 1.0 / (cnt + 1e-8)

    def drain_body(l, _):
        pltpu.make_async_copy(w_hbm.at[ids_tv.at[0]], acc, sem).wait()
        return 0

    lax.fori_loop(0, L, drain_body, 0)

    def scale_body(b, _):
        iv = jnp.full((16,), inv_v[pl.ds(b, 16)][0])
        for d in range(4):
            acc[b, pl.ds(d * 16, 16)] = acc[b, pl.ds(d * 16, 16)] * iv
        return 0

    lax.fori_loop(0, BPW, scale_body, 0)
    pltpu.sync_copy(acc, out_hbm.at[pl.ds(wid * BPW, BPW)])


V = 100000
_TC = 7168
_TCG = HALF_V // _TC


def _tc_transpose_body(xa_ref, xb_ref, y_ref):
    y_ref[:, 0:D] = xa_ref[...].T
    y_ref[:, D : 2 * D] = xb_ref[...].T


def _tc_transpose(wt):
    return pl.pallas_call(
        _tc_transpose_body,
        grid=(_TCG,),
        in_specs=[
            pl.BlockSpec((D, _TC), lambda j: (0, j)),
            pl.BlockSpec((D, _TC), lambda j: (0, j + _TCG)),
        ],
        out_specs=pl.BlockSpec((_TC, 2 * D), lambda j: (j, 0)),
        out_shape=jax.ShapeDtypeStruct((HALF_V, 2 * D), jnp.float32),
    )(wt, wt)


def kernel(ids, weight):
    ids_t = ids.astype(jnp.int32).T
    weight = _tc_transpose(weight.astype(jnp.float32).T).reshape(2 * HALF_V, D)
    mesh = plsc.VectorSubcoreMesh(core_axis_name="c", subcore_axis_name="s")
    run = functools.partial(
        pl.kernel,
        mesh=mesh,
        compiler_params=pltpu.CompilerParams(
            needs_layout_passes=False, use_tc_tiling_on_sc=False
        ),
        out_type=jax.ShapeDtypeStruct((B, D), jnp.float32),
        scratch_types=[
            pltpu.VMEM((L, BPW), jnp.int32),
            pltpu.VMEM((BPW, D), jnp.float32),
            pltpu.VMEM((BPW + 16,), jnp.float32),
            pltpu.SemaphoreType.DMA,
        ],
    )(_pool_kernel)
    return run(ids_t, weight)
